# Initial kernel scaffold; baseline (speedup 1.0000x reference)
import jax
import jax.numpy as jnp
from jax.experimental import pallas as pl


def kernel(x, edge_index, rel_type, norm, weight):
    raise NotImplementedError("write your pallas kernel here")


# speedup vs baseline: 8.3591x; 8.3591x over previous
import functools

import jax
import jax.numpy as jnp
from jax import lax
from jax.experimental import pallas as pl
from jax.experimental.pallas import tpu as pltpu
from jax.experimental.pallas import tpu_sc as plsc

N_CORES = 2
N_SUBCORES = 16
N_WORKERS = N_CORES * N_SUBCORES
LANES = 16
CHUNK = 128


def _transform_body(x_ref, w_ref, o_ref):
    o_ref[...] = jnp.dot(x_ref[...], w_ref[0],
                         preferred_element_type=jnp.float32)[None]


def _transform(x, weight, bn):
    n, d_in = x.shape
    r, _, d_out = weight.shape
    return pl.pallas_call(
        _transform_body,
        grid=(r, n // bn),
        in_specs=[
            pl.BlockSpec((bn, d_in), lambda ri, ni: (ni, 0)),
            pl.BlockSpec((1, d_in, d_out), lambda ri, ni: (ri, 0, 0)),
        ],
        out_specs=pl.BlockSpec((1, bn, d_out), lambda ri, ni: (ri, ni, 0)),
        out_shape=jax.ShapeDtypeStruct((r, n, d_out), jnp.float32),
    )(x, weight)


def _combine_body(p_ref, o_ref):
    o_ref[...] = p_ref[0] + p_ref[1]


def _combine(parts, bn):
    _, n, d = parts.shape
    return pl.pallas_call(
        _combine_body,
        grid=(n // bn,),
        in_specs=[pl.BlockSpec((2, bn, d), lambda i: (0, i, 0))],
        out_specs=pl.BlockSpec((bn, d), lambda i: (i, 0)),
        out_shape=jax.ShapeDtypeStruct((n, d), jnp.float32),
    )(parts)


def _sc_edge_kernel(n_nodes, n_acc, d, n_chunks):
    epw = n_chunks * CHUNK
    rows_per_sub = n_acc // N_SUBCORES
    full = rows_per_sub // CHUNK
    rem = rows_per_sub - full * CHUNK
    mesh = plsc.VectorSubcoreMesh(core_axis_name="c", subcore_axis_name="s")

    @functools.partial(
        pl.kernel,
        out_type=jax.ShapeDtypeStruct((N_CORES * n_acc, d), jnp.float32),
        mesh=mesh,
        scratch_types=[
            pltpu.VMEM((CHUNK,), jnp.int32),
            pltpu.VMEM((CHUNK,), jnp.int32),
            pltpu.VMEM((CHUNK,), jnp.int32),
            pltpu.VMEM((CHUNK,), jnp.int32),
            pltpu.VMEM((CHUNK,), jnp.float32),
            pltpu.VMEM((CHUNK, 128), jnp.float32),
            pltpu.VMEM_SHARED((n_acc, 128), jnp.float32),
            pltpu.SemaphoreType.DMA,
        ],
        compiler_params=pltpu.CompilerParams(needs_layout_passes=False),
    )
    def sc_kernel(table, srcs, rels, dsts, norms, part,
                  src_v, rel_v, idx_v, dst_v, norm_v, rows_v, acc, sem):
        cid = lax.axis_index("c")
        sid = lax.axis_index("s")
        wid = sid * N_CORES + cid
        base = wid * epw
        r0 = sid * rows_per_sub

        def zero_row(i, c):
            for cc in range(d // LANES):
                rows_v[i, pl.ds(cc * LANES, LANES)] = jnp.zeros(
                    (LANES,), jnp.float32)
            return c
        lax.fori_loop(0, CHUNK, zero_row, 0)
        for kk in range(full):
            pltpu.sync_copy(rows_v, acc.at[pl.ds(r0 + kk * CHUNK, CHUNK)])
        if rem:
            pltpu.sync_copy(rows_v.at[pl.ds(0, rem)],
                            acc.at[pl.ds(r0 + full * CHUNK, rem)])
        plsc.subcore_barrier()

        def chunk_body(kc, c):
            eb = base + kc * CHUNK
            pltpu.sync_copy(srcs.at[pl.ds(eb, CHUNK)], src_v)
            pltpu.sync_copy(rels.at[pl.ds(eb, CHUNK)], rel_v)
            pltpu.sync_copy(dsts.at[pl.ds(eb, CHUNK)], dst_v)
            pltpu.sync_copy(norms.at[pl.ds(eb, CHUNK)], norm_v)
            for j in range(CHUNK // LANES):
                sl = pl.ds(j * LANES, LANES)
                idx_v[sl] = rel_v[sl] * n_nodes + src_v[sl]
            pltpu.async_copy(table.at[idx_v], rows_v, sem).wait()

            def scale_e(e, c2):
                nb = plsc.load_gather(
                    norm_v, [jnp.full((LANES,), e, jnp.int32)])
                for cc in range(d // LANES):
                    csl = pl.ds(cc * LANES, LANES)
                    rows_v[e, csl] = rows_v[e, csl] * nb
                return c2
            lax.fori_loop(0, CHUNK, scale_e, 0)

            pltpu.sync_copy(rows_v, acc.at[dst_v], add=True)
            return c
        lax.fori_loop(0, n_chunks, chunk_body, 0)

        plsc.subcore_barrier()

        o0 = cid * n_acc + r0
        for kk in range(full):
            pltpu.sync_copy(acc.at[pl.ds(r0 + kk * CHUNK, CHUNK)],
                            part.at[pl.ds(o0 + kk * CHUNK, CHUNK)])
        if rem:
            pltpu.sync_copy(acc.at[pl.ds(r0 + full * CHUNK, rem)],
                            part.at[pl.ds(o0 + full * CHUNK, rem)])

    return sc_kernel


def kernel(x, edge_index, rel_type, norm, weight):
    n, _ = x.shape
    r_rel, _, d_out = weight.shape
    e = edge_index.shape[1]

    src = edge_index[0].astype(jnp.int32)
    dst = edge_index[1].astype(jnp.int32)
    rel = rel_type.astype(jnp.int32)
    nrm = norm.reshape(-1).astype(jnp.float32)

    stride = N_WORKERS * CHUNK
    e_pad = ((e + stride - 1) // stride) * stride
    pad = e_pad - e
    if pad:
        src = jnp.concatenate([src, jnp.zeros((pad,), jnp.int32)])
        dst = jnp.concatenate([dst, jnp.zeros((pad,), jnp.int32)])
        rel = jnp.concatenate([rel, jnp.zeros((pad,), jnp.int32)])
        nrm = jnp.concatenate([nrm, jnp.zeros((pad,), jnp.float32)])

    bn = 1000 if n % 1000 == 0 else 8
    table = _transform(x, weight, bn).reshape(r_rel * n, d_out)
    acc_stride = N_SUBCORES * CHUNK
    n_acc = ((n + acc_stride - 1) // acc_stride) * acc_stride
    parts = _sc_edge_kernel(n, n_acc, d_out, e_pad // stride)(
        table, src, rel, dst, nrm)
    out_pad = _combine(parts.reshape(N_CORES, n_acc, d_out), n_acc // 10)
    return out_pad[:n]
